# baseline (speedup 1.0000x reference)
import math

import jax
import jax.numpy as jnp
from jax import lax
from jax.experimental import pallas as pl
from jax.experimental.pallas import tpu as pltpu
from jax.experimental.pallas import tpu_sc as plsc

_NEG_INF = float("-inf")


def _sc_count_body(l_q, l_k, sample_k, qb_size, j_half, idx3, cnt_out,
                   buf_v, idx_v):
    cid = lax.axis_index("c")
    sid = lax.axis_index("s")
    w = sid * 2 + cid
    n_qb = l_q // qb_size
    ones = jnp.ones((16,), jnp.float32)
    zeros16 = jnp.zeros((16,), jnp.float32)
    qiota = lax.iota(jnp.int32, 16)

    for half in range(2):
        j0 = w * (2 * j_half) + half * j_half

        def zero_body(z, _):
            buf_v[pl.ds(pl.multiple_of(z * 16, 16), 16)] = zeros16
            return 0

        lax.fori_loop(0, (j_half * l_q) // 16, zero_body, 0)

        def qb_body(qb, _):
            pltpu.sync_copy(idx3.at[qb], idx_v)

            def s_body(s, _):
                def qv_body(qv, _):
                    q0 = pl.multiple_of(qv * 16, 16)
                    vec = idx_v[s, pl.ds(q0, 16)]
                    inr = (vec >= j0) & (vec < j0 + j_half)
                    row = jnp.where(inr, vec - j0, 0)
                    flat = row * l_q + qb * qb_size + q0 + qiota
                    val = jnp.where(inr, ones, 0.0)
                    plsc.addupdate_scatter(buf_v, [flat], val)
                    return 0

                lax.fori_loop(0, qb_size // 16, qv_body, 0)
                return 0

            lax.fori_loop(0, sample_k, s_body, 0)
            return 0

        lax.fori_loop(0, n_qb, qb_body, 0)
        pltpu.sync_copy(buf_v, cnt_out.at[pl.ds(j0 * l_q, j_half * l_q)])


def _sc_count(idx_t, l_q, l_k):
    sample_k = idx_t.shape[0]
    qb_size = 256
    n_qb = l_q // qb_size
    j_half = l_k // 64
    idx3 = (idx_t.reshape(sample_k, n_qb, qb_size)
            .transpose(1, 0, 2))
    mesh = plsc.VectorSubcoreMesh(core_axis_name="c", subcore_axis_name="s",
                                  num_cores=2, num_subcores=16)
    body = lambda *refs: _sc_count_body(l_q, l_k, sample_k, qb_size, j_half,
                                        *refs)
    cnt_flat = pl.kernel(
        body,
        out_type=jax.ShapeDtypeStruct((l_k * l_q,), jnp.float32),
        mesh=mesh,
        compiler_params=pltpu.CompilerParams(needs_layout_passes=False),
        scratch_types=[
            pltpu.VMEM((j_half * l_q,), jnp.float32),
            pltpu.VMEM((sample_k, qb_size), jnp.int32),
        ],
    )(idx3)
    return cnt_flat.reshape(l_k, l_q)


def _index_sample_t(l_q: int, l_k: int, sample_k: int):
    idx = jax.random.randint(jax.random.key(42), (l_q, sample_k), 0, l_k)
    return idx.T.astype(jnp.int32)


def _attn_body(n_top, sample_k, n_bh, chunk, cnt_ref, mask_ref, q_ref,
               k_ref, v_ref, o_ref, m_ref, oh_ref):
    L = q_ref.shape[1]
    D = q_ref.shape[2]
    i = pl.program_id(0)

    @pl.when(i < n_bh)
    def _compute_m():
        q = q_ref[0]
        k = k_ref[0]
        for c in range(0, L, chunk):
            st = lax.dot_general(k, q[c:c + chunk, :],
                                 (((1,), (1,)), ((), ())),
                                 preferred_element_type=jnp.float32)
            cnt = cnt_ref[:, c:c + chunk]
            mmax = jnp.max(jnp.where(cnt > 0.0, st, _NEG_INF), axis=0,
                           keepdims=True)
            msum = jnp.sum(st * cnt, axis=0, keepdims=True)
            m_ref[pl.ds(i, 1), c:c + chunk] = mmax - msum * (1.0 / L)

    @pl.when(i == n_bh)
    def _topk():
        lane = lax.broadcasted_iota(jnp.int32, (n_bh, L), 1)
        m_cur = m_ref[...]
        for r in range(n_top):
            mx = jnp.max(m_cur, axis=1, keepdims=True)
            idx_r = jnp.min(jnp.where(m_cur == mx, lane, L), axis=1,
                            keepdims=True)
            oh_ref[:, r, :] = (lane == idx_r).astype(jnp.float32)
            m_cur = jnp.where(lane == idx_r, _NEG_INF, m_cur)

    @pl.when(i >= n_bh)
    def _attend():
        h = i - n_bh
        q = q_ref[0]
        k = k_ref[0]
        v = v_ref[0]
        onehot = oh_ref[h]
        q_red = jnp.dot(onehot, q, preferred_element_type=jnp.float32)
        scores = lax.dot_general(q_red, k, (((1,), (1,)), ((), ())),
                                 preferred_element_type=jnp.float32)
        scores = scores * (1.0 / math.sqrt(D))
        scores = jnp.where(mask_ref[...] == 0, _NEG_INF, scores)
        smx = jnp.max(scores, axis=1, keepdims=True)
        e = jnp.exp(scores - smx)
        a = e / jnp.sum(e, axis=1, keepdims=True)
        o_ref[0] = jnp.dot(a, v, preferred_element_type=jnp.float32)


def kernel(queries, keys, values, attn_mask):
    B, L_Q, H, D = queries.shape
    L_K = keys.shape[1]
    factor = 5
    u_part = int(factor * math.ceil(math.log(max(L_K, 1))))
    u = int(factor * math.ceil(math.log(max(L_Q, 1))))
    u_part = max(min(u_part, L_K), 1)
    u = max(min(u, L_Q), 1)
    sample_k = min(u_part, L_K)
    n_top = min(u, L_Q)

    idx_t = _index_sample_t(L_Q, L_K, sample_k)
    mask_i = attn_mask.astype(jnp.int32)
    chunk = 512
    BH = B * H

    cnt = _sc_count(idx_t, L_Q, L_K)

    q_t = jnp.swapaxes(queries, 1, 2).reshape(BH, L_Q, D)
    k_t = jnp.swapaxes(keys, 1, 2).reshape(BH, L_K, D)
    v_t = jnp.swapaxes(values, 1, 2).reshape(BH, L_K, D)

    body = lambda *refs: _attn_body(n_top, sample_k, BH, chunk, *refs)
    out = pl.pallas_call(
        body,
        grid=(2 * BH,),
        in_specs=[
            pl.BlockSpec((L_K, L_Q), lambda i: (0, 0)),
            pl.BlockSpec((1, L_K), lambda i: ((i % BH) // H, 0)),
            pl.BlockSpec((1, L_Q, D), lambda i: (i % BH, 0, 0)),
            pl.BlockSpec((1, L_K, D), lambda i: (i % BH, 0, 0)),
            pl.BlockSpec((1, L_K, D), lambda i: (i % BH, 0, 0)),
        ],
        out_specs=pl.BlockSpec((1, n_top, D), lambda i: (i % BH, 0, 0)),
        out_shape=jax.ShapeDtypeStruct((BH, n_top, D), jnp.float32),
        scratch_shapes=[
            pltpu.VMEM((BH, L_Q), jnp.float32),
            pltpu.VMEM((BH, n_top, L_Q), jnp.float32),
        ],
    )(cnt, mask_i, q_t, k_t, v_t)
    return jnp.swapaxes(out.reshape(B, H, n_top, D), 1, 2)
